# baseline (speedup 1.0000x reference)
import functools

import jax
import jax.numpy as jnp
from jax import lax
from jax.experimental import pallas as pl
from jax.experimental.pallas import tpu as pltpu
from jax.experimental.pallas import tpu_sc as plsc


def _table_body(s, emb_ref, w_ref, b_ref, t_ref):
    t_ref[...] = (
        jnp.sum(emb_ref[...] * w_ref[...], axis=1, keepdims=True) + b_ref[0]
    ) * (1.0 / s)


@functools.lru_cache(maxsize=None)
def _make_table_kernel(v_dim, d_dim, s_len):
    return pl.pallas_call(
        functools.partial(_table_body, s_len),
        out_shape=jax.ShapeDtypeStruct((v_dim, 1), jnp.float32),
        in_specs=[
            pl.BlockSpec(memory_space=pltpu.VMEM),
            pl.BlockSpec(memory_space=pltpu.VMEM),
            pl.BlockSpec(memory_space=pltpu.SMEM),
        ],
        out_specs=pl.BlockSpec(memory_space=pltpu.VMEM),
    )


_UNROLL = 8


@functools.lru_cache(maxsize=None)
def _make_sc_kernel(b_rows, s_len, v_dim):
    info = plsc.get_sparse_core_info()
    nc, ns, lanes = info.num_cores, info.num_subcores, info.num_lanes
    nw = nc * ns
    rows_per_w = b_rows // nw
    group = lanes
    chunk = 8 * group
    n_chunks = rows_per_w // chunk
    gpc = chunk // group
    u = _UNROLL
    n_steps, rem = divmod(s_len, u)

    mesh = plsc.VectorSubcoreMesh(core_axis_name="c", subcore_axis_name="s")

    @functools.partial(
        pl.kernel,
        mesh=mesh,
        out_type=jax.ShapeDtypeStruct((b_rows,), jnp.float32),
        scratch_types=[
            pltpu.VMEM((chunk, s_len), jnp.int32),
            pltpu.VMEM((chunk, s_len), jnp.int32),
            pltpu.VMEM((v_dim,), jnp.float32),
            pltpu.VMEM((rows_per_w,), jnp.float32),
            pltpu.SemaphoreType.DMA,
            pltpu.SemaphoreType.DMA,
        ],
        compiler_params=pltpu.CompilerParams(
            needs_layout_passes=False,
            disable_bounds_checks=True,
            use_tc_tiling_on_sc=True,
        ),
    )
    def sc_kernel(x_hbm, t_hbm, out_hbm, xb0, xb1, t_v, o_v, sem0, sem1):
        wid = lax.axis_index("s") * nc + lax.axis_index("c")
        row0 = wid * rows_per_w
        lane_iota = lax.iota(jnp.int32, lanes)
        zero = jnp.zeros((lanes,), jnp.float32)

        def start(c, buf, sem):
            @pl.when(c < n_chunks)
            def _():
                pltpu.async_copy(
                    x_hbm.at[pl.ds(row0 + c * chunk, chunk), :], buf, sem
                )

        def wait(buf, sem):
            pltpu.make_async_copy(
                x_hbm.at[pl.ds(0, chunk), :], buf, sem
            ).wait()

        n_full = s_len // lanes
        rem2 = s_len % lanes
        idx_mask = lane_iota >= (lanes - rem2)
        fmask = idx_mask.astype(jnp.float32)

        def compute(c, buf):
            def do_group(g, carry):
                base = g * group
                vec = zero
                for j in range(group):
                    r = base + j
                    acc_a = zero
                    acc_b = zero
                    for k in range(0, n_full - 1, 2):
                        xi = buf[r, pl.ds(k * lanes, lanes)]
                        acc_a = acc_a + plsc.load_gather(t_v, [xi])
                        xi = buf[r, pl.ds((k + 1) * lanes, lanes)]
                        acc_b = acc_b + plsc.load_gather(t_v, [xi])
                    for k in range(n_full - (n_full % 2), n_full):
                        xi = buf[r, pl.ds(k * lanes, lanes)]
                        acc_a = acc_a + plsc.load_gather(t_v, [xi])
                    if rem2:
                        xi = buf[r, pl.ds(s_len - lanes, lanes)]
                        xi = jnp.where(idx_mask, xi, 0)
                        tv = plsc.load_gather(t_v, [xi])
                        acc_b = acc_b + tv * fmask
                    rowsum = jnp.sum(acc_a + acc_b)
                    vec = jnp.where(lane_iota == j, rowsum, vec)
                res = 1.0 / (1.0 + jnp.exp(-vec))
                o_v[pl.ds(c * chunk + base, lanes)] = res
                return carry

            lax.fori_loop(0, gpc, do_group, 0)

        start(0, xb0, sem0)
        pltpu.sync_copy(t_hbm, t_v)
        start(1, xb1, sem1)

        def outer(i, carry):
            c0 = 2 * i
            wait(xb0, sem0)
            compute(c0, xb0)
            start(c0 + 2, xb0, sem0)
            wait(xb1, sem1)
            compute(c0 + 1, xb1)
            start(c0 + 3, xb1, sem1)
            return carry

        lax.fori_loop(0, n_chunks // 2, outer, 0)
        pltpu.sync_copy(o_v, out_hbm.at[pl.ds(row0, rows_per_w)])

    return sc_kernel


def kernel(x, emb, W, b):
    b_rows, s_len = x.shape
    v_dim, d_dim = emb.shape
    t = _make_table_kernel(v_dim, d_dim, s_len)(emb, W, b)
    out = _make_sc_kernel(b_rows, s_len, v_dim)(x, t.reshape(-1))
    return out.reshape(b_rows, 1)


# speedup vs baseline: 1.1989x; 1.1989x over previous
import functools

import jax
import jax.numpy as jnp
from jax import lax
from jax.experimental import pallas as pl
from jax.experimental.pallas import tpu as pltpu
from jax.experimental.pallas import tpu_sc as plsc


def _table_body(s, emb_ref, w_ref, b_ref, t_ref):
    t_ref[...] = (
        jnp.sum(emb_ref[...] * w_ref[...], axis=1, keepdims=True) + b_ref[0]
    ) * (1.0 / s)


@functools.lru_cache(maxsize=None)
def _make_table_kernel(v_dim, d_dim, s_len):
    return pl.pallas_call(
        functools.partial(_table_body, s_len),
        out_shape=jax.ShapeDtypeStruct((v_dim, 1), jnp.float32),
        in_specs=[
            pl.BlockSpec(memory_space=pltpu.VMEM),
            pl.BlockSpec(memory_space=pltpu.VMEM),
            pl.BlockSpec(memory_space=pltpu.SMEM),
        ],
        out_specs=pl.BlockSpec(memory_space=pltpu.VMEM),
    )


_UNROLL = 8


@functools.lru_cache(maxsize=None)
def _make_sc_kernel(b_rows, s_len, v_dim):
    info = plsc.get_sparse_core_info()
    nc, ns, lanes = info.num_cores, info.num_subcores, info.num_lanes
    nw = nc * ns
    rows_per_w = b_rows // nw
    group = lanes
    chunk = 8 * group
    n_chunks = rows_per_w // chunk
    gpc = chunk // group
    u = _UNROLL
    n_steps, rem = divmod(s_len, u)

    mesh = plsc.VectorSubcoreMesh(core_axis_name="c", subcore_axis_name="s")

    @functools.partial(
        pl.kernel,
        mesh=mesh,
        out_type=jax.ShapeDtypeStruct((b_rows,), jnp.float32),
        scratch_types=[
            pltpu.VMEM((chunk, s_len), jnp.int32),
            pltpu.VMEM((chunk, s_len), jnp.int32),
            pltpu.VMEM((v_dim,), jnp.float32),
            pltpu.VMEM((rows_per_w,), jnp.float32),
            pltpu.SemaphoreType.DMA,
            pltpu.SemaphoreType.DMA,
        ],
        compiler_params=pltpu.CompilerParams(
            needs_layout_passes=False,
            disable_bounds_checks=True,
            use_tc_tiling_on_sc=True,
        ),
    )
    def sc_kernel(x_hbm, t_hbm, out_hbm, xb0, xb1, t_v, o_v, sem0, sem1):
        wid = lax.axis_index("s") * nc + lax.axis_index("c")
        row0 = wid * rows_per_w
        lane_iota = lax.iota(jnp.int32, lanes)
        zero = jnp.zeros((lanes,), jnp.float32)

        def start(c, buf, sem):
            @pl.when(c < n_chunks)
            def _():
                pltpu.async_copy(
                    x_hbm.at[pl.ds(row0 + c * chunk, chunk), :], buf, sem
                )

        def wait(buf, sem):
            pltpu.make_async_copy(
                x_hbm.at[pl.ds(0, chunk), :], buf, sem
            ).wait()

        n_full = s_len // lanes
        rem2 = s_len % lanes
        idx_mask = lane_iota >= (lanes - rem2)
        fmask = idx_mask.astype(jnp.float32)

        def compute(c, buf):
            def do_group(g, carry):
                base = g * group

                def row_quad(q, vec):
                    for jj in range(4):
                        j = 4 * q + jj
                        r = base + j
                        acc_a = zero
                        acc_b = zero
                        for k in range(0, n_full - 1, 2):
                            xi = buf[r, pl.ds(k * lanes, lanes)]
                            acc_a = acc_a + plsc.load_gather(t_v, [xi])
                            xi = buf[r, pl.ds((k + 1) * lanes, lanes)]
                            acc_b = acc_b + plsc.load_gather(t_v, [xi])
                        for k in range(n_full - (n_full % 2), n_full):
                            xi = buf[r, pl.ds(k * lanes, lanes)]
                            acc_a = acc_a + plsc.load_gather(t_v, [xi])
                        if rem2:
                            xi = buf[r, pl.ds(s_len - lanes, lanes)]
                            xi = jnp.where(idx_mask, xi, 0)
                            tv = plsc.load_gather(t_v, [xi])
                            acc_b = acc_b + tv * fmask
                        rowsum = jnp.sum(acc_a + acc_b)
                        vec = jnp.where(lane_iota == j, rowsum, vec)
                    return vec

                vec = lax.fori_loop(0, group // 4, row_quad, zero)
                res = 1.0 / (1.0 + jnp.exp(-vec))
                o_v[pl.ds(c * chunk + base, lanes)] = res
                return carry

            lax.fori_loop(0, gpc, do_group, 0)

        start(0, xb0, sem0)
        pltpu.sync_copy(t_hbm, t_v)
        start(1, xb1, sem1)

        def outer(i, carry):
            c0 = 2 * i
            wait(xb0, sem0)
            compute(c0, xb0)
            start(c0 + 2, xb0, sem0)
            wait(xb1, sem1)
            compute(c0 + 1, xb1)
            start(c0 + 3, xb1, sem1)
            return carry

        lax.fori_loop(0, n_chunks // 2, outer, 0)
        pltpu.sync_copy(o_v, out_hbm.at[pl.ds(row0, rows_per_w)])

    return sc_kernel


def kernel(x, emb, W, b):
    b_rows, s_len = x.shape
    v_dim, d_dim = emb.shape
    t = _make_table_kernel(v_dim, d_dim, s_len)(emb, W, b)
    out = _make_sc_kernel(b_rows, s_len, v_dim)(x, t.reshape(-1))
    return out.reshape(b_rows, 1)


# speedup vs baseline: 1.2025x; 1.0030x over previous
import functools

import jax
import jax.numpy as jnp
from jax import lax
from jax.experimental import pallas as pl
from jax.experimental.pallas import tpu as pltpu
from jax.experimental.pallas import tpu_sc as plsc


def _table_body(s, emb_ref, w_ref, b_ref, t_ref):
    t_ref[...] = (
        jnp.sum(emb_ref[...] * w_ref[...], axis=1, keepdims=True) + b_ref[0]
    ) * (1.0 / s)


@functools.lru_cache(maxsize=None)
def _make_table_kernel(v_dim, d_dim, s_len):
    return pl.pallas_call(
        functools.partial(_table_body, s_len),
        out_shape=jax.ShapeDtypeStruct((v_dim, 1), jnp.float32),
        in_specs=[
            pl.BlockSpec(memory_space=pltpu.VMEM),
            pl.BlockSpec(memory_space=pltpu.VMEM),
            pl.BlockSpec(memory_space=pltpu.SMEM),
        ],
        out_specs=pl.BlockSpec(memory_space=pltpu.VMEM),
    )


@functools.lru_cache(maxsize=None)
def _make_sc_kernel(b_rows, s_len, v_dim):
    info = plsc.get_sparse_core_info()
    nc, ns, lanes = info.num_cores, info.num_subcores, info.num_lanes
    nw = nc * ns
    rows_per_w = b_rows // nw
    group = lanes
    chunk = 8 * group
    n_chunks = rows_per_w // chunk
    gpc = chunk // group

    mesh = plsc.VectorSubcoreMesh(core_axis_name="c", subcore_axis_name="s")

    @functools.partial(
        pl.kernel,
        mesh=mesh,
        out_type=jax.ShapeDtypeStruct((b_rows,), jnp.float32),
        scratch_types=[
            pltpu.VMEM((chunk, s_len), jnp.int32),
            pltpu.VMEM((chunk, s_len), jnp.int32),
            pltpu.VMEM((v_dim,), jnp.float32),
            pltpu.VMEM((rows_per_w,), jnp.float32),
            pltpu.SemaphoreType.DMA,
            pltpu.SemaphoreType.DMA,
        ],
        compiler_params=pltpu.CompilerParams(needs_layout_passes=False),
    )
    def sc_kernel(x_hbm, t_hbm, out_hbm, xb0, xb1, t_v, o_v, sem0, sem1):
        wid = lax.axis_index("s") * nc + lax.axis_index("c")
        row0 = wid * rows_per_w
        lane_iota = lax.iota(jnp.int32, lanes)
        zero = jnp.zeros((lanes,), jnp.float32)

        def start(c, buf, sem):
            @pl.when(c < n_chunks)
            def _():
                pltpu.async_copy(
                    x_hbm.at[pl.ds(row0 + c * chunk, chunk), :], buf, sem
                )

        def wait(buf, sem):
            pltpu.make_async_copy(
                x_hbm.at[pl.ds(0, chunk), :], buf, sem
            ).wait()

        n_full = s_len // lanes
        rem2 = s_len % lanes
        idx_mask = lane_iota >= (lanes - rem2)
        fmask = idx_mask.astype(jnp.float32)

        def compute(c, buf):
            def do_group(g, carry):
                base = g * group

                def row_quad(q, vec):
                    for jj in range(4):
                        j = 4 * q + jj
                        r = base + j
                        acc_a = zero
                        acc_b = zero
                        for k in range(0, n_full - 1, 2):
                            xi = buf[r, pl.ds(k * lanes, lanes)]
                            acc_a = acc_a + plsc.load_gather(t_v, [xi])
                            xi = buf[r, pl.ds((k + 1) * lanes, lanes)]
                            acc_b = acc_b + plsc.load_gather(t_v, [xi])
                        for k in range(n_full - (n_full % 2), n_full):
                            xi = buf[r, pl.ds(k * lanes, lanes)]
                            acc_a = acc_a + plsc.load_gather(t_v, [xi])
                        if rem2:
                            xi = buf[r, pl.ds(s_len - lanes, lanes)]
                            xi = jnp.where(idx_mask, xi, 0)
                            tv = plsc.load_gather(t_v, [xi])
                            acc_b = acc_b + tv * fmask
                        rowsum = jnp.sum(acc_a + acc_b)
                        vec = jnp.where(lane_iota == j, rowsum, vec)
                    return vec

                vec = lax.fori_loop(0, group // 4, row_quad, zero)
                res = 1.0 / (1.0 + jnp.exp(-vec))
                o_v[pl.ds(c * chunk + base, lanes)] = res
                return carry

            lax.fori_loop(0, gpc, do_group, 0)

        start(0, xb0, sem0)
        pltpu.sync_copy(t_hbm, t_v)
        start(1, xb1, sem1)

        def outer(i, carry):
            c0 = 2 * i
            wait(xb0, sem0)
            compute(c0, xb0)
            start(c0 + 2, xb0, sem0)
            wait(xb1, sem1)
            compute(c0 + 1, xb1)
            start(c0 + 3, xb1, sem1)
            return carry

        lax.fori_loop(0, n_chunks // 2, outer, 0)
        pltpu.sync_copy(o_v, out_hbm.at[pl.ds(row0, rows_per_w)])

    return sc_kernel


def kernel(x, emb, W, b):
    b_rows, s_len = x.shape
    v_dim, d_dim = emb.shape
    t = _make_table_kernel(v_dim, d_dim, s_len)(emb, W, b)
    out = _make_sc_kernel(b_rows, s_len, v_dim)(x, t.reshape(-1))
    return out.reshape(b_rows, 1)
